# Initial kernel scaffold; baseline (speedup 1.0000x reference)
import jax
import jax.numpy as jnp
from jax.experimental import pallas as pl


def kernel(x, Wg, W1, b1, W2, b2):
    raise NotImplementedError("write your pallas kernel here")


# speedup vs baseline: 2.0732x; 2.0732x over previous
import functools

import jax
import jax.numpy as jnp
from jax import lax
from jax.experimental import pallas as pl
from jax.experimental.pallas import tpu as pltpu

_D_MODEL = 1024
_D_FF = 4096
_N_EXPERTS = 8
_EPS = 1e-6

_BLK = 128
_N_TOKENS = 2048
_NB = _N_TOKENS // _BLK + _N_EXPERTS
_N_PAD = _NB * _BLK


def _router_body(x_ref, wg_ref, top1_ref, coef_ref):
    x = x_ref[...]
    wg = wg_ref[...]
    scores = lax.dot_general(x, wg, (((1,), (1,)), ((), ())),
                             preferred_element_type=jnp.float32)
    m = jnp.max(scores, axis=1, keepdims=True)
    lane = lax.broadcasted_iota(jnp.int32, scores.shape, 1)
    top1 = jnp.min(jnp.where(scores >= m, lane, _N_EXPERTS), axis=1,
                   keepdims=True)
    onehot = (lane == top1).astype(jnp.float32)
    masked = scores * onehot
    denom = jnp.sum(masked, axis=0, keepdims=True) + _EPS
    capacity = float(_N_TOKENS)
    coef = jnp.sum(masked / denom, axis=1, keepdims=True) * capacity
    top1_ref[...] = top1
    coef_ref[...] = coef


def _router(x, wg):
    n = x.shape[0]
    return pl.pallas_call(
        _router_body,
        out_shape=(
            jax.ShapeDtypeStruct((n, 1), jnp.int32),
            jax.ShapeDtypeStruct((n, 1), jnp.float32),
        ),
    )(x, wg)


def _ffn_body(meta_ref, x_ref, w1_ref, b1_ref, w2_ref, b2_ref, gate_ref,
              out_ref):
    i = pl.program_id(0)
    nblocks = meta_ref[_NB]

    @pl.when(i < nblocks)
    def _():
        xb = x_ref[...]
        h = lax.dot_general(xb, w1_ref[0], (((1,), (1,)), ((), ())),
                            preferred_element_type=jnp.float32)
        h = jnp.maximum(h + b1_ref[0], 0.0).astype(jnp.bfloat16)
        y = lax.dot_general(h, w2_ref[0], (((1,), (1,)), ((), ())),
                            preferred_element_type=jnp.float32)
        out_ref[...] = (y + b2_ref[0]) * gate_ref[...]

    @pl.when(i >= nblocks)
    def _():
        out_ref[...] = jnp.zeros_like(out_ref)


def _ffn(meta, x_pad, w1, b1, w2, b2, gate_pad):
    grid_spec = pltpu.PrefetchScalarGridSpec(
        num_scalar_prefetch=1,
        grid=(_NB,),
        in_specs=[
            pl.BlockSpec((_BLK, _D_MODEL), lambda i, p: (i, 0)),
            pl.BlockSpec((1, _D_FF, _D_MODEL), lambda i, p: (p[i], 0, 0)),
            pl.BlockSpec((1, 1, _D_FF), lambda i, p: (p[i], 0, 0)),
            pl.BlockSpec((1, _D_MODEL, _D_FF), lambda i, p: (p[i], 0, 0)),
            pl.BlockSpec((1, 1, _D_MODEL), lambda i, p: (p[i], 0, 0)),
            pl.BlockSpec((_BLK, 1), lambda i, p: (i, 0)),
        ],
        out_specs=pl.BlockSpec((_BLK, _D_MODEL), lambda i, p: (i, 0)),
    )
    return pl.pallas_call(
        _ffn_body,
        grid_spec=grid_spec,
        out_shape=jax.ShapeDtypeStruct((_N_PAD, _D_MODEL), jnp.float32),
        compiler_params=pltpu.CompilerParams(
            dimension_semantics=("arbitrary",),
            vmem_limit_bytes=128 * 1024 * 1024,
        ),
    )(meta, x_pad, w1, b1, w2, b2, gate_pad)


@jax.jit
def kernel(x, Wg, W1, b1, W2, b2):
    n = x.shape[0]
    top1, coef = _router(x, Wg)
    top1 = top1[:, 0]
    coef = coef[:, 0]

    onehot = jax.nn.one_hot(top1, _N_EXPERTS, dtype=jnp.int32)
    rank = jnp.cumsum(onehot, axis=0) - onehot
    rank = jnp.sum(rank * onehot, axis=1)
    counts = jnp.sum(onehot, axis=0)
    blocks_per_e = (counts + _BLK - 1) // _BLK
    seg_start_blk = jnp.cumsum(blocks_per_e) - blocks_per_e
    nblocks = jnp.sum(blocks_per_e)
    dest = seg_start_blk[top1] * _BLK + rank

    bids = jnp.arange(_NB, dtype=jnp.int32)
    seg_end_blk = jnp.cumsum(blocks_per_e)
    block_expert = jnp.sum(
        (bids[:, None] >= seg_end_blk[None, :]).astype(jnp.int32), axis=1)
    last_e = jnp.max(jnp.where(counts > 0, jnp.arange(_N_EXPERTS), 0))
    block_expert = jnp.where(bids < nblocks, block_expert, last_e)
    meta = jnp.concatenate(
        [block_expert.astype(jnp.int32),
         nblocks.astype(jnp.int32)[None]])

    x_pad = jnp.zeros((_N_PAD, _D_MODEL), jnp.float32).at[dest].set(x)
    gate_pad = jnp.zeros((_N_PAD, 1), jnp.float32).at[dest, 0].set(coef)

    y_pad = _ffn(meta, x_pad.astype(jnp.bfloat16),
                 W1.astype(jnp.bfloat16), b1[:, None, :],
                 W2.astype(jnp.bfloat16), b2[:, None, :], gate_pad)
    return y_pad[dest]


# speedup vs baseline: 2.5878x; 1.2482x over previous
import functools

import jax
import jax.numpy as jnp
from jax import lax
from jax.experimental import pallas as pl
from jax.experimental.pallas import tpu as pltpu

_D_MODEL = 1024
_D_FF = 4096
_N_EXPERTS = 8
_EPS = 1e-6

_BLK = 128
_N_TOKENS = 2048
_NB = _N_TOKENS // _BLK + _N_EXPERTS
_N_PAD = _NB * _BLK


def _router_body(x_ref, wg_ref, top1_ref, coef_ref):
    x = x_ref[...]
    wg = wg_ref[...]
    scores = lax.dot_general(x, wg, (((1,), (1,)), ((), ())),
                             preferred_element_type=jnp.float32)
    m = jnp.max(scores, axis=1, keepdims=True)
    lane = lax.broadcasted_iota(jnp.int32, scores.shape, 1)
    top1 = jnp.min(jnp.where(scores >= m, lane, _N_EXPERTS), axis=1,
                   keepdims=True)
    onehot = (lane == top1).astype(jnp.float32)
    masked = scores * onehot
    denom = jnp.sum(masked, axis=0, keepdims=True) + _EPS
    capacity = float(_N_TOKENS)
    coef = jnp.sum(masked / denom, axis=1, keepdims=True) * capacity
    top1_ref[...] = top1
    coef_ref[...] = coef


def _router(x, wg):
    n = x.shape[0]
    return pl.pallas_call(
        _router_body,
        out_shape=(
            jax.ShapeDtypeStruct((n, 1), jnp.int32),
            jax.ShapeDtypeStruct((n, 1), jnp.float32),
        ),
    )(x, wg)


def _up_body(meta_ref, x_ref, w1_ref, b1_ref, h_ref):
    i = pl.program_id(0)

    @pl.when(i < meta_ref[_NB])
    def _():
        w1 = w1_ref[0].astype(jnp.bfloat16)
        h = lax.dot_general(x_ref[...], w1, (((1,), (1,)), ((), ())),
                            preferred_element_type=jnp.float32)
        h_ref[...] = jnp.maximum(h + b1_ref[0], 0.0).astype(jnp.bfloat16)


def _down_body(meta_ref, h_ref, w2_ref, b2_ref, gate_ref, out_ref):
    i = pl.program_id(0)
    nblocks = meta_ref[_NB]

    @pl.when(i < nblocks)
    def _():
        w2 = w2_ref[0].astype(jnp.bfloat16)
        y = lax.dot_general(h_ref[...], w2, (((1,), (1,)), ((), ())),
                            preferred_element_type=jnp.float32)
        out_ref[...] = (y + b2_ref[0]) * gate_ref[...]

    @pl.when(i >= nblocks)
    def _():
        out_ref[...] = jnp.zeros_like(out_ref)


_CPARAMS = pltpu.CompilerParams(
    dimension_semantics=("arbitrary",),
    vmem_limit_bytes=100 * 1024 * 1024,
)


def _ffn(meta, x_pad, w1, b1, w2, b2, gate_pad):
    up_spec = pltpu.PrefetchScalarGridSpec(
        num_scalar_prefetch=1,
        grid=(_NB,),
        in_specs=[
            pl.BlockSpec((_BLK, _D_MODEL), lambda i, p: (i, 0)),
            pl.BlockSpec((1, _D_FF, _D_MODEL), lambda i, p: (p[i], 0, 0)),
            pl.BlockSpec((1, 1, _D_FF), lambda i, p: (p[i], 0, 0)),
        ],
        out_specs=pl.BlockSpec((_BLK, _D_FF), lambda i, p: (i, 0)),
    )
    h_pad = pl.pallas_call(
        _up_body,
        grid_spec=up_spec,
        out_shape=jax.ShapeDtypeStruct((_N_PAD, _D_FF), jnp.bfloat16),
        compiler_params=_CPARAMS,
    )(meta, x_pad, w1, b1)

    down_spec = pltpu.PrefetchScalarGridSpec(
        num_scalar_prefetch=1,
        grid=(_NB,),
        in_specs=[
            pl.BlockSpec((_BLK, _D_FF), lambda i, p: (i, 0)),
            pl.BlockSpec((1, _D_MODEL, _D_FF), lambda i, p: (p[i], 0, 0)),
            pl.BlockSpec((1, 1, _D_MODEL), lambda i, p: (p[i], 0, 0)),
            pl.BlockSpec((_BLK, 1), lambda i, p: (i, 0)),
        ],
        out_specs=pl.BlockSpec((_BLK, _D_MODEL), lambda i, p: (i, 0)),
    )
    return pl.pallas_call(
        _down_body,
        grid_spec=down_spec,
        out_shape=jax.ShapeDtypeStruct((_N_PAD, _D_MODEL), jnp.float32),
        compiler_params=_CPARAMS,
    )(meta, h_pad, w2, b2, gate_pad)


@jax.jit
def kernel(x, Wg, W1, b1, W2, b2):
    n = x.shape[0]
    top1, coef = _router(x, Wg)
    top1 = top1[:, 0]
    coef = coef[:, 0]

    onehot = jax.nn.one_hot(top1, _N_EXPERTS, dtype=jnp.int32)
    rank = jnp.cumsum(onehot, axis=0) - onehot
    rank = jnp.sum(rank * onehot, axis=1)
    counts = jnp.sum(onehot, axis=0)
    blocks_per_e = (counts + _BLK - 1) // _BLK
    seg_start_blk = jnp.cumsum(blocks_per_e) - blocks_per_e
    nblocks = jnp.sum(blocks_per_e)
    dest = seg_start_blk[top1] * _BLK + rank

    bids = jnp.arange(_NB, dtype=jnp.int32)
    seg_end_blk = jnp.cumsum(blocks_per_e)
    block_expert = jnp.sum(
        (bids[:, None] >= seg_end_blk[None, :]).astype(jnp.int32), axis=1)
    last_e = jnp.max(jnp.where(counts > 0, jnp.arange(_N_EXPERTS), 0))
    block_expert = jnp.where(bids < nblocks, block_expert, last_e)
    meta = jnp.concatenate(
        [block_expert.astype(jnp.int32),
         nblocks.astype(jnp.int32)[None]])

    x_pad = jnp.zeros((_N_PAD, _D_MODEL), jnp.bfloat16).at[dest].set(
        x.astype(jnp.bfloat16))
    gate_pad = jnp.zeros((_N_PAD, 1), jnp.float32).at[dest, 0].set(coef)

    y_pad = _ffn(meta, x_pad, W1, b1[:, None, :], W2, b2[:, None, :],
                 gate_pad)
    return y_pad[dest]


# speedup vs baseline: 2.6565x; 1.0266x over previous
import functools

import jax
import jax.numpy as jnp
from jax import lax
from jax.experimental import pallas as pl
from jax.experimental.pallas import tpu as pltpu

_D_MODEL = 1024
_D_FF = 4096
_N_EXPERTS = 8
_EPS = 1e-6

_BLK = 128
_N_TOKENS = 2048
_NB = _N_TOKENS // _BLK + _N_EXPERTS
_N_PAD = _NB * _BLK


def _router_body(x_ref, wg_ref, top1_ref, coef_ref):
    x = x_ref[...]
    wg = wg_ref[...]
    scores = lax.dot_general(x, wg, (((1,), (1,)), ((), ())),
                             preferred_element_type=jnp.float32)
    m = jnp.max(scores, axis=1, keepdims=True)
    lane = lax.broadcasted_iota(jnp.int32, scores.shape, 1)
    top1 = jnp.min(jnp.where(scores >= m, lane, _N_EXPERTS), axis=1,
                   keepdims=True)
    onehot = (lane == top1).astype(jnp.float32)
    masked = scores * onehot
    denom = jnp.sum(masked, axis=0, keepdims=True) + _EPS
    capacity = float(_N_TOKENS)
    coef = jnp.sum(masked / denom, axis=1, keepdims=True) * capacity
    top1_ref[...] = top1
    coef_ref[...] = coef


def _router(x, wg):
    n = x.shape[0]
    return pl.pallas_call(
        _router_body,
        out_shape=(
            jax.ShapeDtypeStruct((n, 1), jnp.int32),
            jax.ShapeDtypeStruct((n, 1), jnp.float32),
        ),
    )(x, wg)


_FC = 2048
_NF = _D_FF // _FC


def _ffn_body(meta_ref, x_ref, w1_ref, b1_ref, w2_ref, b2_ref, gate_ref,
              out_ref):
    f = pl.program_id(0)
    i = pl.program_id(1)
    rows = pl.ds(i * _BLK, _BLK)

    @pl.when(i < meta_ref[_NB])
    def _():
        w1 = w1_ref[0].astype(jnp.bfloat16)
        h = lax.dot_general(x_ref[...], w1, (((1,), (1,)), ((), ())),
                            preferred_element_type=jnp.float32)
        h = jnp.maximum(h + b1_ref[0], 0.0).astype(jnp.bfloat16)
        w2 = w2_ref[0].astype(jnp.bfloat16)
        part = lax.dot_general(h, w2, (((1,), (1,)), ((), ())),
                               preferred_element_type=jnp.float32)

        @pl.when(f == 0)
        def _():
            out_ref[rows, :] = part

        @pl.when((f > 0) & (f < _NF - 1))
        def _():
            out_ref[rows, :] += part

        @pl.when((f == _NF - 1) & (f > 0))
        def _():
            out_ref[rows, :] = (out_ref[rows, :] + part + b2_ref[0]) \
                * gate_ref[...]


def _ffn(meta, x_pad, w1, b1, w2, b2, gate_pad):
    grid_spec = pltpu.PrefetchScalarGridSpec(
        num_scalar_prefetch=1,
        grid=(_NF, _NB),
        in_specs=[
            pl.BlockSpec((_BLK, _D_MODEL), lambda f, i, p: (i, 0)),
            pl.BlockSpec((1, _FC, _D_MODEL), lambda f, i, p: (p[i], f, 0)),
            pl.BlockSpec((1, 1, _FC), lambda f, i, p: (p[i], 0, f)),
            pl.BlockSpec((1, _D_MODEL, _FC), lambda f, i, p: (p[i], 0, f)),
            pl.BlockSpec((1, 1, _D_MODEL), lambda f, i, p: (p[i], 0, 0)),
            pl.BlockSpec((_BLK, 1), lambda f, i, p: (i, 0)),
        ],
        out_specs=pl.BlockSpec((_N_PAD, _D_MODEL), lambda f, i, p: (0, 0)),
    )
    return pl.pallas_call(
        _ffn_body,
        grid_spec=grid_spec,
        out_shape=jax.ShapeDtypeStruct((_N_PAD, _D_MODEL), jnp.float32),
        compiler_params=pltpu.CompilerParams(
            dimension_semantics=("arbitrary", "arbitrary"),
            vmem_limit_bytes=100 * 1024 * 1024,
        ),
    )(meta, x_pad, w1, b1, w2, b2, gate_pad)


@jax.jit
def kernel(x, Wg, W1, b1, W2, b2):
    n = x.shape[0]
    top1, coef = _router(x, Wg)
    top1 = top1[:, 0]
    coef = coef[:, 0]

    onehot = jax.nn.one_hot(top1, _N_EXPERTS, dtype=jnp.int32)
    rank = jnp.cumsum(onehot, axis=0) - onehot
    rank = jnp.sum(rank * onehot, axis=1)
    counts = jnp.sum(onehot, axis=0)
    blocks_per_e = (counts + _BLK - 1) // _BLK
    seg_start_blk = jnp.cumsum(blocks_per_e) - blocks_per_e
    nblocks = jnp.sum(blocks_per_e)
    dest = seg_start_blk[top1] * _BLK + rank

    bids = jnp.arange(_NB, dtype=jnp.int32)
    seg_end_blk = jnp.cumsum(blocks_per_e)
    block_expert = jnp.sum(
        (bids[:, None] >= seg_end_blk[None, :]).astype(jnp.int32), axis=1)
    last_e = jnp.max(jnp.where(counts > 0, jnp.arange(_N_EXPERTS), 0))
    block_expert = jnp.where(bids < nblocks, block_expert, last_e)
    meta = jnp.concatenate(
        [block_expert.astype(jnp.int32),
         nblocks.astype(jnp.int32)[None]])

    x_pad = jnp.zeros((_N_PAD, _D_MODEL), jnp.bfloat16).at[dest].set(
        x.astype(jnp.bfloat16))
    gate_pad = jnp.zeros((_N_PAD, 1), jnp.float32).at[dest, 0].set(coef)

    y_pad = _ffn(meta, x_pad, W1, b1[:, None, :], W2, b2[:, None, :],
                 gate_pad)
    return y_pad[dest]


# speedup vs baseline: 2.8501x; 1.0729x over previous
import functools

import jax
import jax.numpy as jnp
from jax import lax
from jax.experimental import pallas as pl
from jax.experimental.pallas import tpu as pltpu

_D_MODEL = 1024
_D_FF = 4096
_N_EXPERTS = 8
_EPS = 1e-6

_BLK = 128
_N_TOKENS = 2048
_NB = _N_TOKENS // _BLK + _N_EXPERTS
_N_PAD = _NB * _BLK


_CHUNK = 256


def _router_body(x_ref, wg_ref, dest_ref, coef_ref, meta_ref, xbf_ref):
    x = x_ref[...]
    wg = wg_ref[...]
    n = x.shape[0]
    scores = lax.dot_general(x, wg, (((1,), (1,)), ((), ())),
                             preferred_element_type=jnp.float32)
    m = jnp.max(scores, axis=1, keepdims=True)
    lane = lax.broadcasted_iota(jnp.int32, scores.shape, 1)
    top1 = jnp.min(jnp.where(scores >= m, lane, _N_EXPERTS), axis=1,
                   keepdims=True)
    onehot = (lane == top1).astype(jnp.float32)
    masked = scores * onehot
    denom = jnp.sum(masked, axis=0, keepdims=True) + _EPS
    capacity = float(_N_TOKENS)
    coef = jnp.sum(masked / denom, axis=1, keepdims=True) * capacity

    cum_rows = []
    for c in range(n // _CHUNK):
        r_ids = lax.broadcasted_iota(jnp.int32, (_CHUNK, n), 0) \
            + c * _CHUNK
        c_ids = lax.broadcasted_iota(jnp.int32, (_CHUNK, n), 1)
        tri = (c_ids <= r_ids).astype(jnp.float32)
        cum_rows.append(
            lax.dot_general(tri, onehot, (((1,), (0,)), ((), ())),
                            preferred_element_type=jnp.float32))
    cum = jnp.concatenate(cum_rows, axis=0)
    rank = jnp.sum(cum * onehot, axis=1, keepdims=True) - 1.0
    counts = cum[n - 1:n, :].astype(jnp.int32)
    bpe = (counts + _BLK - 1) // _BLK

    er = lax.broadcasted_iota(jnp.int32, (_N_EXPERTS, _N_EXPERTS), 0)
    ec = lax.broadcasted_iota(jnp.int32, (_N_EXPERTS, _N_EXPERTS), 1)
    strict_up = (er < ec).astype(jnp.float32)
    seg_start = lax.dot_general(bpe.astype(jnp.float32), strict_up,
                                (((1,), (0,)), ((), ())),
                                preferred_element_type=jnp.float32)
    nblocks = jnp.sum(bpe, axis=1, keepdims=True)
    seg_end = seg_start.astype(jnp.int32) + bpe

    dest_f = jnp.sum(onehot * seg_start, axis=1, keepdims=True) * \
        float(_BLK) + rank
    dest_ref[...] = dest_f.astype(jnp.int32)
    coef_ref[...] = coef

    jb = lax.broadcasted_iota(jnp.int32, (_NB, _N_EXPERTS), 0)
    be = jnp.sum((jb >= seg_end).astype(jnp.int32), axis=1,
                 keepdims=True)
    lane8 = lax.broadcasted_iota(jnp.int32, (1, _N_EXPERTS), 1)
    last_e = jnp.max(jnp.where(counts > 0, lane8, 0), axis=1,
                     keepdims=True)
    jcol = lax.broadcasted_iota(jnp.int32, (_NB, 1), 0)
    be = jnp.where(jcol < nblocks, be, last_e)
    meta_ref[...] = jnp.concatenate([be, nblocks], axis=0)
    xbf_ref[...] = x.astype(jnp.bfloat16)


def _router(x, wg):
    n = x.shape[0]
    return pl.pallas_call(
        _router_body,
        out_shape=(
            jax.ShapeDtypeStruct((n, 1), jnp.int32),
            jax.ShapeDtypeStruct((n, 1), jnp.float32),
            jax.ShapeDtypeStruct((_NB + 1, 1), jnp.int32),
            jax.ShapeDtypeStruct((n, _D_MODEL), jnp.bfloat16),
        ),
        compiler_params=pltpu.CompilerParams(
            vmem_limit_bytes=100 * 1024 * 1024,
        ),
    )(x, wg)


_FC = 2048
_NF = _D_FF // _FC


def _ffn_body(meta_ref, x_ref, w1_ref, b1_ref, w2_ref, b2_ref, out_ref):
    f = pl.program_id(0)
    i = pl.program_id(1)
    rows = pl.ds(i * _BLK, _BLK)

    @pl.when(i < meta_ref[_NB, 0])
    def _():
        w1 = w1_ref[0].astype(jnp.bfloat16)
        h = lax.dot_general(x_ref[...], w1, (((1,), (1,)), ((), ())),
                            preferred_element_type=jnp.float32)
        h = jnp.maximum(h + b1_ref[0], 0.0).astype(jnp.bfloat16)
        w2 = w2_ref[0].astype(jnp.bfloat16)
        part = lax.dot_general(h, w2, (((1,), (1,)), ((), ())),
                               preferred_element_type=jnp.float32)

        @pl.when(f == 0)
        def _():
            out_ref[rows, :] = part

        @pl.when((f > 0) & (f < _NF - 1))
        def _():
            out_ref[rows, :] += part

        @pl.when((f == _NF - 1) & (f > 0))
        def _():
            out_ref[rows, :] = out_ref[rows, :] + part + b2_ref[0]


def _ffn(meta, x_pad, w1, b1, w2, b2):
    grid_spec = pltpu.PrefetchScalarGridSpec(
        num_scalar_prefetch=1,
        grid=(_NF, _NB),
        in_specs=[
            pl.BlockSpec((_BLK, _D_MODEL), lambda f, i, p: (i, 0)),
            pl.BlockSpec((1, _FC, _D_MODEL), lambda f, i, p: (p[i, 0], f, 0)),
            pl.BlockSpec((1, 1, _FC), lambda f, i, p: (p[i, 0], 0, f)),
            pl.BlockSpec((1, _D_MODEL, _FC), lambda f, i, p: (p[i, 0], 0, f)),
            pl.BlockSpec((1, 1, _D_MODEL), lambda f, i, p: (p[i, 0], 0, 0)),
        ],
        out_specs=pl.BlockSpec((_N_PAD, _D_MODEL), lambda f, i, p: (0, 0)),
    )
    return pl.pallas_call(
        _ffn_body,
        grid_spec=grid_spec,
        out_shape=jax.ShapeDtypeStruct((_N_PAD, _D_MODEL), jnp.float32),
        compiler_params=pltpu.CompilerParams(
            dimension_semantics=("arbitrary", "arbitrary"),
            vmem_limit_bytes=100 * 1024 * 1024,
        ),
    )(meta, x_pad, w1, b1, w2, b2)


@jax.jit
def kernel(x, Wg, W1, b1, W2, b2):
    dest2, coef, meta, xbf = _router(x, Wg)
    dest = dest2[:, 0]

    x_pad = jnp.zeros((_N_PAD, _D_MODEL), jnp.bfloat16).at[dest].set(xbf)
    y_pad = _ffn(meta, x_pad, W1, b1[:, None, :], W2, b2[:, None, :])
    return y_pad[dest] * coef
